# baseline (speedup 1.0000x reference)
import jax
import jax.numpy as jnp
from jax import lax
from jax.experimental import pallas as pl
from jax.experimental.pallas import tpu as pltpu
from jax.experimental.pallas import tpu_sc as plsc

_RADIUS = 0.2
_NSAMPLE = 32

_B = 4
_N = 8192
_P = 1024
_C = 64
_COUT = 3 + _C
_TW = 128

_NC = 2
_NS = 16
_NW = _NC * _NS
_CEN_PER_W = (_B * _P) // _NW
_P_PER_W = _P // (_NW // _B)
_G = 8
_NG = _CEN_PER_W // _G
_NPAIR = _NG // 2
_GS = _G * _NSAMPLE
_IDXCAP = _NSAMPLE + 256 + 16


def _sc_body(table_hbm, xyzf_hbm, newf_hbm, out_hbm,
             xr, yr, zr, qxr, qyr, qzr, xyzi, idxbuf, ptrbuf,
             idxblk, parbuf, locbuf, rows, tilebuf, ptr_s,
             sem_g0, sem_g1, sem_o):
    wid = lax.axis_index("s") * _NC + lax.axis_index("c")
    b = wid // (_NW // _B)
    p0 = (wid % (_NW // _B)) * _P_PER_W
    gbase = wid * _CEN_PER_W

    lane = lax.iota(jnp.int32, 16)
    l3 = lane * 3
    r2 = jnp.float32(_RADIUS * _RADIUS)
    tb4 = b * (_N // 2)
    sems = (sem_g0, sem_g1)

    def stage_pts(ch, cr):
        pltpu.sync_copy(xyzf_hbm.at[pl.ds(b * (_N * 3) + ch * 3072, 3072)],
                        xyzi)

        def deint(k, c2):
            base = pl.multiple_of(ch * 1024 + k * 16, 16)
            iv = l3 + k * 48
            xr[pl.ds(base, 16)] = plsc.load_gather(xyzi, [iv])
            yr[pl.ds(base, 16)] = plsc.load_gather(xyzi, [iv + 1])
            zr[pl.ds(base, 16)] = plsc.load_gather(xyzi, [iv + 2])
            return c2

        return lax.fori_loop(0, 64, deint, cr)

    lax.fori_loop(0, _N // 1024, stage_pts, jnp.int32(0))

    pltpu.sync_copy(newf_hbm.at[pl.ds(gbase * 3, _CEN_PER_W * 3)],
                    xyzi.at[pl.ds(0, _CEN_PER_W * 3)])

    def deint_cen(k, cr):
        base = pl.multiple_of(k * 16, 16)
        iv = l3 + k * 48
        qxr[pl.ds(base, 16)] = plsc.load_gather(xyzi, [iv])
        qyr[pl.ds(base, 16)] = plsc.load_gather(xyzi, [iv + 1])
        qzr[pl.ds(base, 16)] = plsc.load_gather(xyzi, [iv + 2])
        return cr

    lax.fori_loop(0, _CEN_PER_W // 16, deint_cen, jnp.int32(0))

    def q_group(g, par):
        pb = par * _GS

        def qf_body(i, cr):
            cen = g * _G + i
            qxv = jnp.full((16,), qxr[pl.ds(cen, 16)][0])
            qyv = jnp.full((16,), qyr[pl.ds(cen, 16)][0])
            qzv = jnp.full((16,), qzr[pl.ds(cen, 16)][0])

            ptrbuf[pl.ds(0, 16)] = jnp.zeros((16,), jnp.int32)
            ptr_s[0] = jnp.int32(0)

            def superchunk(sc_i, c2):
                @pl.when(ptr_s[0] < _NSAMPLE)
                def _go():
                    def chunk_body(ci, ptrv):
                        base = pl.multiple_of((sc_i * 8 + ci) * 32, 32)
                        dxa = xr[pl.ds(base, 16)] - qxv
                        dya = yr[pl.ds(base, 16)] - qyv
                        dza = zr[pl.ds(base, 16)] - qzv
                        dxb = xr[pl.ds(base + 16, 16)] - qxv
                        dyb = yr[pl.ds(base + 16, 16)] - qyv
                        dzb = zr[pl.ds(base + 16, 16)] - qzv
                        d2a = (dxa * dxa + dya * dya) + dza * dza
                        d2b = (dxb * dxb + dyb * dyb) + dzb * dzb
                        ma = d2a < r2
                        mb = d2b < r2
                        ra = plsc.cumsum(ma.astype(jnp.int32))
                        rb_ = plsc.cumsum(mb.astype(jnp.int32))
                        ca = plsc.all_reduce_population_count(ma)
                        cb = plsc.all_reduce_population_count(mb)
                        pm1 = ptrv - 1
                        plsc.store_scatter(idxbuf, [pm1 + ra],
                                           lane + base, mask=ma)
                        pm2 = pm1 + ca
                        plsc.store_scatter(idxbuf, [pm2 + rb_],
                                           lane + (base + 16), mask=mb)
                        return ptrv + (ca + cb)
                    pv = lax.fori_loop(0, 8, chunk_body,
                                       ptrbuf[pl.ds(0, 16)], unroll=8)
                    ptrbuf[pl.ds(0, 16)] = pv
                    ptr_s[0] = pv[0]
                return c2

            lax.fori_loop(0, _N // 256, superchunk, jnp.int32(0))
            nf = ptr_s[0]

            raw0 = idxbuf[pl.ds(0, 16)]
            first = jnp.where(nf > jnp.int32(0), raw0[0], jnp.int32(0))
            fv = jnp.full((16,), first)
            v0 = jnp.where(lane < nf, raw0, fv)
            v1 = jnp.where(lane + 16 < nf, idxbuf[pl.ds(16, 16)], fv)

            rb = pb + i * _NSAMPLE
            locbuf[pl.ds(rb, 16)] = v0
            locbuf[pl.ds(rb + 16, 16)] = v1
            idxblk[pl.ds(rb, 16)] = tb4 + lax.shift_right_logical(v0, 1)
            idxblk[pl.ds(rb + 16, 16)] = tb4 + lax.shift_right_logical(v1, 1)
            parbuf[pl.ds(rb, 16)] = lax.shift_left(v0 & 1, 6)
            parbuf[pl.ds(rb + 16, 16)] = lax.shift_left(v1 & 1, 6)

            return cr

        lax.fori_loop(0, _G, qf_body, jnp.int32(0))
        pltpu.async_copy(table_hbm.at[idxblk.at[pl.ds(pb, _GS)]],
                         rows.at[pl.ds(pb, _GS)], sems[par])

    def t_group(g, par):
        pb = par * _GS
        pltpu.make_async_copy(table_hbm.at[pl.ds(0, _GS)],
                              rows.at[pl.ds(pb, _GS)], sems[par]).wait()

        def tp_body(i, cr):
            cen = g * _G + i
            qxv = jnp.full((16,), qxr[pl.ds(cen, 16)][0])
            qyv = jnp.full((16,), qyr[pl.ds(cen, 16)][0])
            qzv = jnp.full((16,), qzr[pl.ds(cen, 16)][0])
            rb = pb + i * _NSAMPLE
            lidx0 = locbuf[pl.ds(rb, 16)]
            lidx1 = locbuf[pl.ds(rb + 16, 16)]
            po0 = parbuf[pl.ds(rb, 16)]
            po1 = parbuf[pl.ds(rb + 16, 16)]
            rs0 = rb + lane
            rs1 = rb + 16 + lane
            cb = i * _NSAMPLE
            for c in range(_COUT):
                if c < 3:
                    src = (xr, yr, zr)[c]
                    qv = (qxv, qyv, qzv)[c]
                    g0 = plsc.load_gather(src, [lidx0]) - qv
                    g1 = plsc.load_gather(src, [lidx1]) - qv
                else:
                    cc = jnp.full((16,), jnp.int32(c - 3))
                    g0 = plsc.load_gather(rows, [rs0, po0 + cc])
                    g1 = plsc.load_gather(rows, [rs1, po1 + cc])
                tilebuf[par, c, pl.ds(cb, 16)] = g0
                tilebuf[par, c, pl.ds(cb + 16, 16)] = g1
            return cr

        lax.fori_loop(0, _G, tp_body, jnp.int32(0))
        colbase = (p0 + g * _G) * _NSAMPLE
        pltpu.async_copy(tilebuf.at[par],
                         out_hbm.at[b, :, pl.ds(colbase, _GS)], sem_o)

    def drain_out(par):
        pltpu.make_async_copy(out_hbm.at[b, :, pl.ds(0, _GS)],
                              tilebuf.at[par], sem_o).wait()

    q_group(jnp.int32(0), 0)

    def pair_body(gp, carry):
        g_even = gp * 2
        q_group(g_even + 1, 1)

        @pl.when(gp >= 1)
        def _d0():
            drain_out(0)
        t_group(g_even, 0)

        @pl.when(gp < _NPAIR - 1)
        def _qnext():
            q_group(g_even + 2, 0)

        @pl.when(gp >= 1)
        def _d1():
            drain_out(1)
        t_group(g_even + 1, 1)
        return carry

    lax.fori_loop(0, _NPAIR, pair_body, jnp.int32(0))
    drain_out(0)
    drain_out(1)


@jax.jit
def _qag_sc(table, xyz_flat, new_flat):
    mesh = plsc.VectorSubcoreMesh(core_axis_name="c", subcore_axis_name="s")
    return pl.kernel(
        _sc_body,
        out_type=jax.ShapeDtypeStruct((_B, _COUT, _P * _NSAMPLE),
                                      jnp.float32),
        mesh=mesh,
        compiler_params=pltpu.CompilerParams(needs_layout_passes=False,
                                             use_tc_tiling_on_sc=False,
                                             disable_bounds_checks=True),
        scratch_types=[
            pltpu.VMEM((_N,), jnp.float32),
            pltpu.VMEM((_N,), jnp.float32),
            pltpu.VMEM((_N,), jnp.float32),
            pltpu.VMEM((_CEN_PER_W + 16,), jnp.float32),
            pltpu.VMEM((_CEN_PER_W + 16,), jnp.float32),
            pltpu.VMEM((_CEN_PER_W + 16,), jnp.float32),
            pltpu.VMEM((3072,), jnp.float32),
            pltpu.VMEM((_IDXCAP,), jnp.int32),
            pltpu.VMEM((16,), jnp.int32),
            pltpu.VMEM((2 * _GS,), jnp.int32),
            pltpu.VMEM((2 * _GS,), jnp.int32),
            pltpu.VMEM((2 * _GS,), jnp.int32),
            pltpu.VMEM((2 * _GS, _TW), jnp.float32),
            pltpu.VMEM((2, _COUT, _GS), jnp.float32),
            pltpu.SMEM((1,), jnp.int32),
            pltpu.SemaphoreType.DMA,
            pltpu.SemaphoreType.DMA,
            pltpu.SemaphoreType.DMA,
        ],
    )(table, xyz_flat, new_flat)


def kernel(xyz, new_xyz, features):
    table = jnp.transpose(features, (0, 2, 1)).reshape(_B * _N // 2, _TW)
    out = _qag_sc(table, xyz.reshape(_B * _N * 3),
                  new_xyz.reshape(_B * _P * 3))
    return out.reshape(_B, _COUT, _P, _NSAMPLE)


# speedup vs baseline: 1.0011x; 1.0011x over previous
import jax
import jax.numpy as jnp
from jax import lax
from jax.experimental import pallas as pl
from jax.experimental.pallas import tpu as pltpu
from jax.experimental.pallas import tpu_sc as plsc

_RADIUS = 0.2
_NSAMPLE = 32

_B = 4
_N = 8192
_P = 1024
_C = 64
_COUT = 3 + _C
_TW = 128

_NC = 2
_NS = 16
_NW = _NC * _NS
_CEN_PER_W = (_B * _P) // _NW
_P_PER_W = _P // (_NW // _B)
_G = 4
_NG = _CEN_PER_W // _G
_NPAIR = _NG // 2
_GS = _G * _NSAMPLE
_IDXCAP = _NSAMPLE + 256 + 16


def _sc_body(table_hbm, xyzf_hbm, newf_hbm, out_hbm,
             xr, yr, zr, qxr, qyr, qzr, xyzi, idxbuf, ptrbuf,
             idxblk, parbuf, locbuf, rows, tilebuf, ptr_s,
             sem_g0, sem_g1, sem_o):
    wid = lax.axis_index("s") * _NC + lax.axis_index("c")
    b = wid // (_NW // _B)
    p0 = (wid % (_NW // _B)) * _P_PER_W
    gbase = wid * _CEN_PER_W

    lane = lax.iota(jnp.int32, 16)
    l3 = lane * 3
    r2 = jnp.float32(_RADIUS * _RADIUS)
    tb4 = b * (_N // 2)
    sems = (sem_g0, sem_g1)

    def stage_pts(ch, cr):
        pltpu.sync_copy(xyzf_hbm.at[pl.ds(b * (_N * 3) + ch * 3072, 3072)],
                        xyzi)

        def deint(k, c2):
            base = pl.multiple_of(ch * 1024 + k * 16, 16)
            iv = l3 + k * 48
            xr[pl.ds(base, 16)] = plsc.load_gather(xyzi, [iv])
            yr[pl.ds(base, 16)] = plsc.load_gather(xyzi, [iv + 1])
            zr[pl.ds(base, 16)] = plsc.load_gather(xyzi, [iv + 2])
            return c2

        return lax.fori_loop(0, 64, deint, cr)

    lax.fori_loop(0, _N // 1024, stage_pts, jnp.int32(0))

    pltpu.sync_copy(newf_hbm.at[pl.ds(gbase * 3, _CEN_PER_W * 3)],
                    xyzi.at[pl.ds(0, _CEN_PER_W * 3)])

    def deint_cen(k, cr):
        base = pl.multiple_of(k * 16, 16)
        iv = l3 + k * 48
        qxr[pl.ds(base, 16)] = plsc.load_gather(xyzi, [iv])
        qyr[pl.ds(base, 16)] = plsc.load_gather(xyzi, [iv + 1])
        qzr[pl.ds(base, 16)] = plsc.load_gather(xyzi, [iv + 2])
        return cr

    lax.fori_loop(0, _CEN_PER_W // 16, deint_cen, jnp.int32(0))

    def q_group(g, par):
        pb = par * _GS

        def qf_body(i, cr):
            cen = g * _G + i
            qxv = jnp.full((16,), qxr[pl.ds(cen, 16)][0])
            qyv = jnp.full((16,), qyr[pl.ds(cen, 16)][0])
            qzv = jnp.full((16,), qzr[pl.ds(cen, 16)][0])

            ptrbuf[pl.ds(0, 16)] = jnp.zeros((16,), jnp.int32)
            ptr_s[0] = jnp.int32(0)

            def superchunk(sc_i, c2):
                @pl.when(ptr_s[0] < _NSAMPLE)
                def _go():
                    def chunk_body(ci, ptrv):
                        base = pl.multiple_of((sc_i * 8 + ci) * 32, 32)
                        dxa = xr[pl.ds(base, 16)] - qxv
                        dya = yr[pl.ds(base, 16)] - qyv
                        dza = zr[pl.ds(base, 16)] - qzv
                        dxb = xr[pl.ds(base + 16, 16)] - qxv
                        dyb = yr[pl.ds(base + 16, 16)] - qyv
                        dzb = zr[pl.ds(base + 16, 16)] - qzv
                        d2a = (dxa * dxa + dya * dya) + dza * dza
                        d2b = (dxb * dxb + dyb * dyb) + dzb * dzb
                        ma = d2a < r2
                        mb = d2b < r2
                        ra = plsc.cumsum(ma.astype(jnp.int32))
                        rb_ = plsc.cumsum(mb.astype(jnp.int32))
                        ca = plsc.all_reduce_population_count(ma)
                        cb = plsc.all_reduce_population_count(mb)
                        pm1 = ptrv - 1
                        plsc.store_scatter(idxbuf, [pm1 + ra],
                                           lane + base, mask=ma)
                        pm2 = pm1 + ca
                        plsc.store_scatter(idxbuf, [pm2 + rb_],
                                           lane + (base + 16), mask=mb)
                        return ptrv + (ca + cb)
                    pv = lax.fori_loop(0, 8, chunk_body,
                                       ptrbuf[pl.ds(0, 16)], unroll=8)
                    ptrbuf[pl.ds(0, 16)] = pv
                    ptr_s[0] = pv[0]
                return c2

            lax.fori_loop(0, _N // 256, superchunk, jnp.int32(0))
            nf = ptr_s[0]

            raw0 = idxbuf[pl.ds(0, 16)]
            first = jnp.where(nf > jnp.int32(0), raw0[0], jnp.int32(0))
            fv = jnp.full((16,), first)
            v0 = jnp.where(lane < nf, raw0, fv)
            v1 = jnp.where(lane + 16 < nf, idxbuf[pl.ds(16, 16)], fv)

            rb = pb + i * _NSAMPLE
            locbuf[pl.ds(rb, 16)] = v0
            locbuf[pl.ds(rb + 16, 16)] = v1
            idxblk[pl.ds(rb, 16)] = tb4 + lax.shift_right_logical(v0, 1)
            idxblk[pl.ds(rb + 16, 16)] = tb4 + lax.shift_right_logical(v1, 1)
            parbuf[pl.ds(rb, 16)] = lax.shift_left(v0 & 1, 6)
            parbuf[pl.ds(rb + 16, 16)] = lax.shift_left(v1 & 1, 6)

            return cr

        lax.fori_loop(0, _G, qf_body, jnp.int32(0))
        pltpu.async_copy(table_hbm.at[idxblk.at[pl.ds(pb, _GS)]],
                         rows.at[pl.ds(pb, _GS)], sems[par])

    def t_group(g, par):
        pb = par * _GS
        pltpu.make_async_copy(table_hbm.at[pl.ds(0, _GS)],
                              rows.at[pl.ds(pb, _GS)], sems[par]).wait()

        def tp_body(i, cr):
            cen = g * _G + i
            qxv = jnp.full((16,), qxr[pl.ds(cen, 16)][0])
            qyv = jnp.full((16,), qyr[pl.ds(cen, 16)][0])
            qzv = jnp.full((16,), qzr[pl.ds(cen, 16)][0])
            rb = pb + i * _NSAMPLE
            lidx0 = locbuf[pl.ds(rb, 16)]
            lidx1 = locbuf[pl.ds(rb + 16, 16)]
            po0 = parbuf[pl.ds(rb, 16)]
            po1 = parbuf[pl.ds(rb + 16, 16)]
            rs0 = rb + lane
            rs1 = rb + 16 + lane
            cb = i * _NSAMPLE
            for c in range(_COUT):
                if c < 3:
                    src = (xr, yr, zr)[c]
                    qv = (qxv, qyv, qzv)[c]
                    g0 = plsc.load_gather(src, [lidx0]) - qv
                    g1 = plsc.load_gather(src, [lidx1]) - qv
                else:
                    cc = jnp.full((16,), jnp.int32(c - 3))
                    g0 = plsc.load_gather(rows, [rs0, po0 + cc])
                    g1 = plsc.load_gather(rows, [rs1, po1 + cc])
                tilebuf[par, c, pl.ds(cb, 16)] = g0
                tilebuf[par, c, pl.ds(cb + 16, 16)] = g1
            return cr

        lax.fori_loop(0, _G, tp_body, jnp.int32(0))
        colbase = (p0 + g * _G) * _NSAMPLE
        pltpu.async_copy(tilebuf.at[par],
                         out_hbm.at[b, :, pl.ds(colbase, _GS)], sem_o)

    def drain_out(par):
        pltpu.make_async_copy(out_hbm.at[b, :, pl.ds(0, _GS)],
                              tilebuf.at[par], sem_o).wait()

    q_group(jnp.int32(0), 0)

    def pair_body(gp, carry):
        g_even = gp * 2
        q_group(g_even + 1, 1)

        @pl.when(gp >= 1)
        def _d0():
            drain_out(0)
        t_group(g_even, 0)

        @pl.when(gp < _NPAIR - 1)
        def _qnext():
            q_group(g_even + 2, 0)

        @pl.when(gp >= 1)
        def _d1():
            drain_out(1)
        t_group(g_even + 1, 1)
        return carry

    lax.fori_loop(0, _NPAIR, pair_body, jnp.int32(0))
    drain_out(0)
    drain_out(1)


@jax.jit
def _qag_sc(table, xyz_flat, new_flat):
    mesh = plsc.VectorSubcoreMesh(core_axis_name="c", subcore_axis_name="s")
    return pl.kernel(
        _sc_body,
        out_type=jax.ShapeDtypeStruct((_B, _COUT, _P * _NSAMPLE),
                                      jnp.float32),
        mesh=mesh,
        compiler_params=pltpu.CompilerParams(needs_layout_passes=False,
                                             use_tc_tiling_on_sc=False,
                                             disable_bounds_checks=True),
        scratch_types=[
            pltpu.VMEM((_N,), jnp.float32),
            pltpu.VMEM((_N,), jnp.float32),
            pltpu.VMEM((_N,), jnp.float32),
            pltpu.VMEM((_CEN_PER_W + 16,), jnp.float32),
            pltpu.VMEM((_CEN_PER_W + 16,), jnp.float32),
            pltpu.VMEM((_CEN_PER_W + 16,), jnp.float32),
            pltpu.VMEM((3072,), jnp.float32),
            pltpu.VMEM((_IDXCAP,), jnp.int32),
            pltpu.VMEM((16,), jnp.int32),
            pltpu.VMEM((2 * _GS,), jnp.int32),
            pltpu.VMEM((2 * _GS,), jnp.int32),
            pltpu.VMEM((2 * _GS,), jnp.int32),
            pltpu.VMEM((2 * _GS, _TW), jnp.float32),
            pltpu.VMEM((2, _COUT, _GS), jnp.float32),
            pltpu.SMEM((1,), jnp.int32),
            pltpu.SemaphoreType.DMA,
            pltpu.SemaphoreType.DMA,
            pltpu.SemaphoreType.DMA,
        ],
    )(table, xyz_flat, new_flat)


def kernel(xyz, new_xyz, features):
    table = jnp.transpose(features, (0, 2, 1)).reshape(_B * _N // 2, _TW)
    out = _qag_sc(table, xyz.reshape(_B * _N * 3),
                  new_xyz.reshape(_B * _P * 3))
    return out.reshape(_B, _COUT, _P, _NSAMPLE)
